# baseline (speedup 1.0000x reference)
import functools

import jax
import jax.numpy as jnp
from jax import lax
from jax.experimental import pallas as pl
from jax.experimental.pallas import tpu as pltpu
from jax.experimental.pallas import tpu_sc as plsc

N = 10000
E = 320000
D = 128
DE = 16
EPS = 1e-5

_NC = 2
_NS = 16
_NW = _NC * _NS

_K = 16
_EW = E // _NW
_NCHUNK = _EW // _K
_M = 10112
_HOLE = _M // 2
_RPT = _M // _NS
_TP = N + DE


def _tables_body(x_ref, wf_ref, ws_ref, t2_ref):
    x = x_ref[...]
    wf = wf_ref[...]
    ws = ws_ref[...]
    dn = (((1,), (1,)), ((), ()))
    f32 = jnp.float32
    af = lax.dot_general(x, wf[:, :D], dn, preferred_element_type=f32)
    as_ = lax.dot_general(x, ws[:, :D], dn, preferred_element_type=f32)
    bf_ = lax.dot_general(x, wf[:, D:2 * D], dn, preferred_element_type=f32)
    bs_ = lax.dot_general(x, ws[:, D:2 * D], dn, preferred_element_type=f32)
    t2_ref[0:_HOLE, 0:D] = af[0:_HOLE]
    t2_ref[0:_HOLE, D:] = as_[0:_HOLE]
    t2_ref[_HOLE + 8:N + 8, 0:D] = af[_HOLE:N]
    t2_ref[_HOLE + 8:N + 8, D:] = as_[_HOLE:N]
    t2_ref[_TP:_TP + N, 0:D] = bf_
    t2_ref[_TP:_TP + N, D:] = bs_


def _node_tables(x, Wf, Ws):
    return pl.pallas_call(
        _tables_body,
        out_shape=jax.ShapeDtypeStruct((_TP + N, 2 * D), jnp.float32),
    )(x, Wf, Ws)


_BE = 8000


def _epack_body(ea_ref, wfe_ref, wse_ref, bf_ref, bs_ref, out_ref):
    ea = ea_ref[...]
    dn = (((1,), (1,)), ((), ()))
    f32 = jnp.float32
    out_ref[:, :D] = lax.dot_general(ea, wfe_ref[...], dn, preferred_element_type=f32) + bf_ref[...]
    out_ref[:, D:] = lax.dot_general(ea, wse_ref[...], dn, preferred_element_type=f32) + bs_ref[...]


def _edge_tables(edge_attr, Wfe, Wse, bf, bs):
    grid = E // _BE
    return pl.pallas_call(
        _epack_body,
        grid=(grid,),
        in_specs=[
            pl.BlockSpec((_BE, DE), lambda i: (i, 0)),
            pl.BlockSpec((D, DE), lambda i: (0, 0)),
            pl.BlockSpec((D, DE), lambda i: (0, 0)),
            pl.BlockSpec((1, D), lambda i: (0, 0)),
            pl.BlockSpec((1, D), lambda i: (0, 0)),
        ],
        out_specs=pl.BlockSpec((_BE, 2 * D), lambda i: (i, 0)),
        out_shape=jax.ShapeDtypeStruct((E, 2 * D), jnp.float32),
    )(edge_attr, Wfe, Wse, bf.reshape(1, D), bs.reshape(1, D))


def _sc_edge_body(t2_hbm, epack_hbm, ip_hbm, dm_hbm, z_hbm, out_hbm,
                  ip_v, dm_v, buf_g, buf_e, msg,
                  agg_sh, sem_i, sem_g):
    c = lax.axis_index("c")
    s = lax.axis_index("s")
    row0 = pl.multiple_of(s * _RPT, 8)
    pltpu.sync_copy(z_hbm.at[pl.ds(row0, _RPT)], agg_sh.at[pl.ds(row0, _RPT)])
    plsc.subcore_barrier()
    w = c * _NS + s
    base0 = w * _EW

    def start_idx(ci, b):
        base = pl.multiple_of(base0 + ci * _K, 8)
        pltpu.async_copy(ip_hbm.at[pl.ds(base * 2, 2 * _K)], ip_v[b], sem_i[b])
        pltpu.async_copy(dm_hbm.at[pl.ds(base, _K)], dm_v[b], sem_i[b])

    def wait_idx(b):
        pltpu.make_async_copy(ip_hbm.at[pl.ds(0, 2 * _K)], ip_v[b], sem_i[b]).wait()
        pltpu.make_async_copy(dm_hbm.at[pl.ds(0, _K)], dm_v[b], sem_i[b]).wait()

    def start_gathers(ci, b):
        base = pl.multiple_of(base0 + ci * _K, 8)
        pltpu.async_copy(t2_hbm.at[ip_v[b]], buf_g[b], sem_g[b])
        pltpu.async_copy(epack_hbm.at[pl.ds(base, _K)], buf_e[b], sem_g[b])

    def wait_gathers(b):
        pltpu.make_async_copy(t2_hbm.at[pl.ds(0, 2 * _K)], buf_g[b], sem_g[b]).wait()
        pltpu.make_async_copy(epack_hbm.at[pl.ds(0, _K)], buf_e[b], sem_g[b]).wait()

    def compute(b):
        def row(r, rcarry):
            for j in range(D // 16):
                cf = 16 * j
                cs = D + 16 * j
                pf = buf_g[b][r, pl.ds(cf, 16)] + buf_g[b][_K + r, pl.ds(cf, 16)] + buf_e[b][r, pl.ds(cf, 16)]
                ps = buf_g[b][r, pl.ds(cs, 16)] + buf_g[b][_K + r, pl.ds(cs, 16)] + buf_e[b][r, pl.ds(cs, 16)]
                sig = 1.0 / (1.0 + jnp.exp(-pf))
                u = jnp.exp(-jnp.abs(ps))
                p = -0.0064535442
                p = p * u + 0.0360884937
                p = p * u + -0.0953293897
                p = p * u + 0.1676540711
                p = p * u + -0.2407338084
                p = p * u + 0.3317990258
                p = p * u + -0.4998741238
                p = p * u + 0.9999964239
                sp = jnp.maximum(ps, 0.0) + p * u
                msg[b][r, pl.ds(cf, 16)] = sig * sp
            return rcarry

        lax.fori_loop(0, _K, row, 0)

    start_idx(0, 0)
    wait_idx(0)
    start_gathers(0, 0)
    start_idx(1, 1)
    start_idx(2, 2)

    def triple(t, carry):
        for b in range(3):
            ch = 3 * t + b
            b1 = (b + 1) % 3
            wait_idx(b1)
            start_gathers(ch + 1, b1)
            wait_gathers(b)
            compute(b)
            pltpu.sync_copy(msg[b], agg_sh.at[dm_v[b]], add=True)

            @pl.when(ch + 3 < _NCHUNK)
            def _():
                start_idx(ch + 3, b)

        return carry

    lax.fori_loop(0, (_NCHUNK - 1) // 3, triple, 0)
    wait_gathers(0)
    compute(0)
    pltpu.sync_copy(msg[0], agg_sh.at[dm_v[0]], add=True)

    plsc.subcore_barrier()
    pltpu.sync_copy(agg_sh.at[pl.ds(row0, _RPT)],
                    out_hbm.at[pl.ds(pl.multiple_of(c * _M + row0, 8), _RPT)])


def _edge_phase_sc(t2, epack, ipack, dstm, zeros):
    mesh = plsc.VectorSubcoreMesh(core_axis_name="c", subcore_axis_name="s")
    run = pl.kernel(
        _sc_edge_body,
        out_type=jax.ShapeDtypeStruct((_NC * _M, D), jnp.float32),
        mesh=mesh,
        scratch_types=[
            [pltpu.VMEM((2 * _K,), jnp.int32)] * 3,
            [pltpu.VMEM((_K,), jnp.int32)] * 3,
            [pltpu.VMEM((2 * _K, 2 * D), jnp.float32)] * 3,
            [pltpu.VMEM((_K, 2 * D), jnp.float32)] * 3,
            [pltpu.VMEM((_K, D), jnp.float32)] * 3,
            pltpu.VMEM_SHARED((_M, D), jnp.float32),
            [pltpu.SemaphoreType.DMA] * 3,
            [pltpu.SemaphoreType.DMA] * 3,
        ],
    )
    return run(t2, epack, ipack, dstm, zeros)


def _post_body(p_ref, x_ref, bng_ref, bnb_ref, lng_ref, lnb_ref, o_ref):
    agg_top = p_ref[0:_HOLE] + p_ref[_M:_M + _HOLE]
    agg_bot = p_ref[_HOLE + 8:N + 8] + p_ref[_M + _HOLE + 8:_M + N + 8]
    agg = jnp.concatenate([agg_top, agg_bot], axis=0)
    x = x_ref[...]
    mean = jnp.mean(agg, axis=0, keepdims=True)
    d = agg - mean
    var = jnp.mean(d * d, axis=0, keepdims=True)
    agg_bn = d * lax.rsqrt(var + EPS) * bng_ref[...] + bnb_ref[...]
    conv = agg_bn + x
    mu = jnp.mean(conv, axis=1, keepdims=True)
    dd = conv - mu
    v = jnp.mean(dd * dd, axis=1, keepdims=True)
    h = dd * lax.rsqrt(v + EPS) * lng_ref[...] + lnb_ref[...]
    o_ref[...] = jnp.maximum(h, 0.0) + x


def _postprocess(partials, x, bn_gamma, bn_beta, ln_gamma, ln_beta):
    return pl.pallas_call(
        _post_body,
        out_shape=jax.ShapeDtypeStruct((N, D), jnp.float32),
    )(partials, x, bn_gamma.reshape(1, D), bn_beta.reshape(1, D),
      ln_gamma.reshape(1, D), ln_beta.reshape(1, D))


def kernel(x, edge_index, edge_attr, Wf, bf, Ws, bs, bn_gamma, bn_beta, ln_gamma, ln_beta):
    src = edge_index[0].astype(jnp.int32)
    dst = edge_index[1].astype(jnp.int32)
    dstm = dst + 8 * (dst >= _HOLE).astype(jnp.int32)
    ipack = jnp.stack(
        [dstm.reshape(-1, _K), (src + _TP).reshape(-1, _K)], axis=1).reshape(-1)
    zeros = jnp.zeros((_M, D), jnp.float32)
    t2 = _node_tables(x, Wf, Ws)
    epack = _edge_tables(edge_attr, Wf[:, 2 * D:], Ws[:, 2 * D:], bf, bs)
    partials = _edge_phase_sc(t2, epack, ipack, dstm, zeros)
    return _postprocess(partials, x, bn_gamma, bn_beta, ln_gamma, ln_beta)


# speedup vs baseline: 2.1585x; 2.1585x over previous
import functools

import jax
import jax.numpy as jnp
from jax import lax
from jax.experimental import pallas as pl
from jax.experimental.pallas import tpu as pltpu
from jax.experimental.pallas import tpu_sc as plsc

N = 10000
E = 320000
D = 128
DE = 16
EPS = 1e-5

_NC = 2
_NS = 16
_NW = _NC * _NS

_K = 16
_EW = E // _NW
_NCHUNK = _EW // _K
_M = 10112
_HOLE = _M // 2
_RPT = _M // _NS
_TP = N + DE


def _tables_body(x_ref, wf_ref, ws_ref, t2_ref):
    x = x_ref[...]
    wf = wf_ref[...]
    ws = ws_ref[...]
    dn = (((1,), (1,)), ((), ()))
    f32 = jnp.float32
    af = lax.dot_general(x, wf[:, :D], dn, preferred_element_type=f32)
    as_ = lax.dot_general(x, ws[:, :D], dn, preferred_element_type=f32)
    bf_ = lax.dot_general(x, wf[:, D:2 * D], dn, preferred_element_type=f32)
    bs_ = lax.dot_general(x, ws[:, D:2 * D], dn, preferred_element_type=f32)
    t2_ref[0:_HOLE, 0:D] = af[0:_HOLE]
    t2_ref[0:_HOLE, D:] = as_[0:_HOLE]
    t2_ref[_HOLE + 8:N + 8, 0:D] = af[_HOLE:N]
    t2_ref[_HOLE + 8:N + 8, D:] = as_[_HOLE:N]
    t2_ref[_TP:_TP + N, 0:D] = bf_
    t2_ref[_TP:_TP + N, D:] = bs_


def _node_tables(x, Wf, Ws):
    return pl.pallas_call(
        _tables_body,
        out_shape=jax.ShapeDtypeStruct((_TP + N, 2 * D), jnp.float32),
    )(x, Wf, Ws)


_BE = 8000


def _epack_body(ea_ref, wfe_ref, wse_ref, bf_ref, bs_ref, out_ref):
    ea = ea_ref[...]
    dn = (((1,), (1,)), ((), ()))
    f32 = jnp.float32
    out_ref[:, :D] = lax.dot_general(ea, wfe_ref[...], dn, preferred_element_type=f32) + bf_ref[...]
    out_ref[:, D:] = lax.dot_general(ea, wse_ref[...], dn, preferred_element_type=f32) + bs_ref[...]


def _edge_tables(edge_attr, Wfe, Wse, bf, bs):
    grid = E // _BE
    return pl.pallas_call(
        _epack_body,
        grid=(grid,),
        in_specs=[
            pl.BlockSpec((_BE, DE), lambda i: (i, 0)),
            pl.BlockSpec((D, DE), lambda i: (0, 0)),
            pl.BlockSpec((D, DE), lambda i: (0, 0)),
            pl.BlockSpec((1, D), lambda i: (0, 0)),
            pl.BlockSpec((1, D), lambda i: (0, 0)),
        ],
        out_specs=pl.BlockSpec((_BE, 2 * D), lambda i: (i, 0)),
        out_shape=jax.ShapeDtypeStruct((E, 2 * D), jnp.float32),
    )(edge_attr, Wfe, Wse, bf.reshape(1, D), bs.reshape(1, D))


def _sc_edge_body(t2_hbm, epack_hbm, ip_hbm, dm_hbm, z_hbm, out_hbm,
                  ip_v, dm_v, buf_g, buf_e, msg,
                  agg_sh, sem_i, sem_g):
    c = lax.axis_index("c")
    s = lax.axis_index("s")
    row0 = pl.multiple_of(s * _RPT, 8)
    pltpu.sync_copy(z_hbm.at[pl.ds(row0, _RPT)], agg_sh.at[pl.ds(row0, _RPT)])
    plsc.subcore_barrier()
    w = c * _NS + s
    base0 = w * _EW

    def start_idx(ci, b):
        base = pl.multiple_of(base0 + ci * _K, 8)
        pltpu.async_copy(ip_hbm.at[pl.ds(base * 2, 2 * _K)], ip_v[b], sem_i[b])
        pltpu.async_copy(dm_hbm.at[pl.ds(base, _K)], dm_v[b], sem_i[b])

    def wait_idx(b):
        pltpu.make_async_copy(ip_hbm.at[pl.ds(0, 2 * _K)], ip_v[b], sem_i[b]).wait()
        pltpu.make_async_copy(dm_hbm.at[pl.ds(0, _K)], dm_v[b], sem_i[b]).wait()

    def start_gathers(ci, b):
        base = pl.multiple_of(base0 + ci * _K, 8)
        pltpu.async_copy(t2_hbm.at[ip_v[b]], buf_g[b], sem_g[b])
        pltpu.async_copy(epack_hbm.at[pl.ds(base, _K)], buf_e[b], sem_g[b])

    def wait_gathers(b):
        pltpu.make_async_copy(t2_hbm.at[pl.ds(0, 2 * _K)], buf_g[b], sem_g[b]).wait()
        pltpu.make_async_copy(epack_hbm.at[pl.ds(0, _K)], buf_e[b], sem_g[b]).wait()

    def compute(b):
        @plsc.parallel_loop(0, _K, 1, unroll=2)
        def row(r):
            for j in range(D // 16):
                cf = 16 * j
                cs = D + 16 * j
                pf = buf_g[b][r, pl.ds(cf, 16)] + buf_g[b][_K + r, pl.ds(cf, 16)] + buf_e[b][r, pl.ds(cf, 16)]
                ps = buf_g[b][r, pl.ds(cs, 16)] + buf_g[b][_K + r, pl.ds(cs, 16)] + buf_e[b][r, pl.ds(cs, 16)]
                sig = 1.0 / (1.0 + jnp.exp(-pf))
                u = jnp.exp(-jnp.abs(ps))
                t = u / (u + 2.0)
                t2 = t * t
                log1p = 2.0 * t * (1.0 + t2 * (1.0 / 3.0 + t2 * 0.2))
                sp = jnp.maximum(ps, 0.0) + log1p
                msg[b][r, pl.ds(cf, 16)] = sig * sp

    start_idx(0, 0)
    wait_idx(0)
    start_gathers(0, 0)
    start_idx(1, 1)
    start_idx(2, 2)

    def triple(t, carry):
        for b in range(3):
            ch = 3 * t + b
            b1 = (b + 1) % 3
            wait_idx(b1)
            start_gathers(ch + 1, b1)
            wait_gathers(b)
            compute(b)
            pltpu.sync_copy(msg[b], agg_sh.at[dm_v[b]], add=True)

            @pl.when(ch + 3 < _NCHUNK)
            def _():
                start_idx(ch + 3, b)

        return carry

    lax.fori_loop(0, (_NCHUNK - 1) // 3, triple, 0)
    wait_gathers(0)
    compute(0)
    pltpu.sync_copy(msg[0], agg_sh.at[dm_v[0]], add=True)

    plsc.subcore_barrier()
    pltpu.sync_copy(agg_sh.at[pl.ds(row0, _RPT)],
                    out_hbm.at[pl.ds(pl.multiple_of(c * _M + row0, 8), _RPT)])


def _edge_phase_sc(t2, epack, ipack, dstm, zeros):
    mesh = plsc.VectorSubcoreMesh(core_axis_name="c", subcore_axis_name="s")
    run = pl.kernel(
        _sc_edge_body,
        out_type=jax.ShapeDtypeStruct((_NC * _M, D), jnp.float32),
        mesh=mesh,
        scratch_types=[
            [pltpu.VMEM((2 * _K,), jnp.int32)] * 3,
            [pltpu.VMEM((_K,), jnp.int32)] * 3,
            [pltpu.VMEM((2 * _K, 2 * D), jnp.float32)] * 3,
            [pltpu.VMEM((_K, 2 * D), jnp.float32)] * 3,
            [pltpu.VMEM((_K, D), jnp.float32)] * 3,
            pltpu.VMEM_SHARED((_M, D), jnp.float32),
            [pltpu.SemaphoreType.DMA] * 3,
            [pltpu.SemaphoreType.DMA] * 3,
        ],
    )
    return run(t2, epack, ipack, dstm, zeros)


def _post_body(p_ref, x_ref, bng_ref, bnb_ref, lng_ref, lnb_ref, o_ref):
    agg_top = p_ref[0:_HOLE] + p_ref[_M:_M + _HOLE]
    agg_bot = p_ref[_HOLE + 8:N + 8] + p_ref[_M + _HOLE + 8:_M + N + 8]
    agg = jnp.concatenate([agg_top, agg_bot], axis=0)
    x = x_ref[...]
    mean = jnp.mean(agg, axis=0, keepdims=True)
    d = agg - mean
    var = jnp.mean(d * d, axis=0, keepdims=True)
    agg_bn = d * lax.rsqrt(var + EPS) * bng_ref[...] + bnb_ref[...]
    conv = agg_bn + x
    mu = jnp.mean(conv, axis=1, keepdims=True)
    dd = conv - mu
    v = jnp.mean(dd * dd, axis=1, keepdims=True)
    h = dd * lax.rsqrt(v + EPS) * lng_ref[...] + lnb_ref[...]
    o_ref[...] = jnp.maximum(h, 0.0) + x


def _postprocess(partials, x, bn_gamma, bn_beta, ln_gamma, ln_beta):
    return pl.pallas_call(
        _post_body,
        out_shape=jax.ShapeDtypeStruct((N, D), jnp.float32),
    )(partials, x, bn_gamma.reshape(1, D), bn_beta.reshape(1, D),
      ln_gamma.reshape(1, D), ln_beta.reshape(1, D))


def kernel(x, edge_index, edge_attr, Wf, bf, Ws, bs, bn_gamma, bn_beta, ln_gamma, ln_beta):
    src = edge_index[0].astype(jnp.int32)
    dst = edge_index[1].astype(jnp.int32)
    dstm = dst + 8 * (dst >= _HOLE).astype(jnp.int32)
    ipack = jnp.stack(
        [dstm.reshape(-1, _K), (src + _TP).reshape(-1, _K)], axis=1).reshape(-1)
    zeros = jnp.zeros((_M, D), jnp.float32)
    t2 = _node_tables(x, Wf, Ws)
    epack = _edge_tables(edge_attr, Wf[:, 2 * D:], Ws[:, 2 * D:], bf, bs)
    partials = _edge_phase_sc(t2, epack, ipack, dstm, zeros)
    return _postprocess(partials, x, bn_gamma, bn_beta, ln_gamma, ln_beta)
